# speedup vs baseline: 32.4851x; 32.4851x over previous
import jax
import jax.numpy as jnp
from jax import lax
from jax.experimental import pallas as pl
from jax.experimental.pallas import tpu as pltpu

_K = 16
_INV1 = 1.0 / (2.0 * (8 * 0.05) ** 2)
_INV2 = 1.0 / (2.0 * (4 * 0.05) ** 2)


def _interp_weights(dxyz, sxyz, inv_two_bw2):
    a2 = jnp.sum(dxyz * dxyz, axis=0)[:, None]
    b2 = jnp.sum(sxyz * sxyz, axis=0)[None, :]
    ab = lax.dot_general(dxyz, sxyz, (((0,), (0,)), ((), ())))
    d2 = jnp.maximum(a2 + b2 - 2.0 * ab, 0.0)

    def body(_, cur):
        return jnp.min(jnp.where(d2 > cur, d2, jnp.inf), axis=1, keepdims=True)

    thr = lax.fori_loop(0, _K, body, jnp.full((d2.shape[0], 1), -jnp.inf, d2.dtype))

    w = jnp.where(d2 <= thr, jnp.exp(d2 * (-inv_two_bw2)), 0.0)
    return w / (jnp.sum(w, axis=1, keepdims=True) + 1e-8)


def _fused_kernel(l2x_ref, l2p_ref, l3x_ref, l3p_ref, l4x_ref, l4p_ref,
                  W1_ref, b1_ref, W2_ref, b2_ref,
                  W3_ref, b3_ref, W4_ref, b4_ref, out_ref):
    w_a = _interp_weights(l3x_ref[0], l4x_ref[0], _INV1)
    interp = lax.dot_general(l4p_ref[0], w_a, (((1,), (1,)), ((), ())))
    new = interp + l3p_ref[0]
    h = jnp.maximum(jnp.dot(W1_ref[...], new) + b1_ref[...], 0.0)
    l3_new = jnp.maximum(jnp.dot(W2_ref[...], h) + b2_ref[...], 0.0)

    w_b = _interp_weights(l2x_ref[0], l3x_ref[0], _INV2)
    interp2 = lax.dot_general(l3_new, w_b, (((1,), (1,)), ((), ())))
    new2 = interp2 + l2p_ref[0]
    h2 = jnp.maximum(jnp.dot(W3_ref[...], new2) + b3_ref[...], 0.0)
    out_ref[0] = jnp.maximum(jnp.dot(W4_ref[...], h2) + b4_ref[...], 0.0)


def kernel(l1_xyz, l1_points, l2_xyz, l2_points, l3_xyz, l3_points,
           l4_xyz, l4_points, W1, b1, W2, b2, W3, b3, W4, b4):
    del l1_xyz, l1_points
    B = l2_xyz.shape[0]
    Nd2, Nd3, Nd4 = l2_xyz.shape[2], l3_xyz.shape[2], l4_xyz.shape[2]
    C = l2_points.shape[1]
    Cout = W4.shape[0]

    def batch_spec(shape):
        return pl.BlockSpec((1,) + shape, lambda b: (b, 0, 0))

    def full_spec(shape):
        return pl.BlockSpec(shape, lambda b: (0,) * len(shape))

    b1c = b1.reshape(-1, 1)
    b2c = b2.reshape(-1, 1)
    b3c = b3.reshape(-1, 1)
    b4c = b4.reshape(-1, 1)

    out = pl.pallas_call(
        _fused_kernel,
        grid=(B,),
        in_specs=[
            batch_spec((3, Nd2)), batch_spec((C, Nd2)),
            batch_spec((3, Nd3)), batch_spec((C, Nd3)),
            batch_spec((3, Nd4)), batch_spec((C, Nd4)),
            full_spec(W1.shape), full_spec(b1c.shape),
            full_spec(W2.shape), full_spec(b2c.shape),
            full_spec(W3.shape), full_spec(b3c.shape),
            full_spec(W4.shape), full_spec(b4c.shape),
        ],
        out_specs=batch_spec((Cout, Nd2)),
        out_shape=jax.ShapeDtypeStruct((B, Cout, Nd2), l2_points.dtype),
        compiler_params=pltpu.CompilerParams(
            dimension_semantics=("arbitrary",),
        ),
    )(l2_xyz, l2_points, l3_xyz, l3_points, l4_xyz, l4_points,
      W1, b1c, W2, b2c, W3, b3c, W4, b4c)
    return out


# speedup vs baseline: 51.1729x; 1.5753x over previous
import jax
import jax.numpy as jnp
from jax import lax
from jax.experimental import pallas as pl
from jax.experimental.pallas import tpu as pltpu

_K = 16
_INV1 = 1.0 / (2.0 * (8 * 0.05) ** 2)
_INV2 = 1.0 / (2.0 * (4 * 0.05) ** 2)


def _interp_weights_t(sxyz, dxyz, inv_two_bw2):
    b2 = jnp.sum(sxyz * sxyz, axis=0)[:, None]
    a2 = jnp.sum(dxyz * dxyz, axis=0)[None, :]
    ab = lax.dot_general(sxyz, dxyz, (((0,), (0,)), ((), ())))
    d2 = jnp.maximum(b2 + a2 - 2.0 * ab, 0.0)

    cur = jnp.full((1, d2.shape[1]), -jnp.inf, d2.dtype)
    for _ in range(_K):
        cur = jnp.min(jnp.where(d2 > cur, d2, jnp.inf), axis=0, keepdims=True)

    w = jnp.where(d2 <= cur, jnp.exp(d2 * (-inv_two_bw2)), 0.0)
    return w / (jnp.sum(w, axis=0, keepdims=True) + 1e-8)


def _fused_kernel(l2x_ref, l2p_ref, l3x_ref, l3p_ref, l4x_ref, l4p_ref,
                  W1_ref, b1_ref, W2_ref, b2_ref,
                  W3_ref, b3_ref, W4_ref, b4_ref, out_ref):
    w_a = _interp_weights_t(l4x_ref[0], l3x_ref[0], _INV1)
    w_b = _interp_weights_t(l3x_ref[0], l2x_ref[0], _INV2)

    interp = jnp.dot(l4p_ref[0], w_a)
    new = interp + l3p_ref[0]
    h = jnp.maximum(jnp.dot(W1_ref[...], new) + b1_ref[...], 0.0)
    l3_new = jnp.maximum(jnp.dot(W2_ref[...], h) + b2_ref[...], 0.0)

    interp2 = jnp.dot(l3_new, w_b)
    new2 = interp2 + l2p_ref[0]
    h2 = jnp.maximum(jnp.dot(W3_ref[...], new2) + b3_ref[...], 0.0)
    out_ref[0] = jnp.maximum(jnp.dot(W4_ref[...], h2) + b4_ref[...], 0.0)


def kernel(l1_xyz, l1_points, l2_xyz, l2_points, l3_xyz, l3_points,
           l4_xyz, l4_points, W1, b1, W2, b2, W3, b3, W4, b4):
    del l1_xyz, l1_points
    B = l2_xyz.shape[0]
    Nd2, Nd3, Nd4 = l2_xyz.shape[2], l3_xyz.shape[2], l4_xyz.shape[2]
    C = l2_points.shape[1]
    Cout = W4.shape[0]

    def batch_spec(shape):
        return pl.BlockSpec((1,) + shape, lambda b: (b, 0, 0))

    def full_spec(shape):
        return pl.BlockSpec(shape, lambda b: (0,) * len(shape))

    b1c = b1.reshape(-1, 1)
    b2c = b2.reshape(-1, 1)
    b3c = b3.reshape(-1, 1)
    b4c = b4.reshape(-1, 1)

    out = pl.pallas_call(
        _fused_kernel,
        grid=(B,),
        in_specs=[
            batch_spec((3, Nd2)), batch_spec((C, Nd2)),
            batch_spec((3, Nd3)), batch_spec((C, Nd3)),
            batch_spec((3, Nd4)), batch_spec((C, Nd4)),
            full_spec(W1.shape), full_spec(b1c.shape),
            full_spec(W2.shape), full_spec(b2c.shape),
            full_spec(W3.shape), full_spec(b3c.shape),
            full_spec(W4.shape), full_spec(b4c.shape),
        ],
        out_specs=batch_spec((Cout, Nd2)),
        out_shape=jax.ShapeDtypeStruct((B, Cout, Nd2), l2_points.dtype),
        compiler_params=pltpu.CompilerParams(
            dimension_semantics=("parallel",),
        ),
    )(l2_xyz, l2_points, l3_xyz, l3_points, l4_xyz, l4_points,
      W1, b1c, W2, b2c, W3, b3c, W4, b4c)
    return out


# speedup vs baseline: 60.9736x; 1.1915x over previous
import jax
import jax.numpy as jnp
from jax import lax
from jax.experimental import pallas as pl
from jax.experimental.pallas import tpu as pltpu

_K = 16
_INV1 = 1.0 / (2.0 * (8 * 0.05) ** 2)
_INV2 = 1.0 / (2.0 * (4 * 0.05) ** 2)


def _interp_weights_t(sxyz, dxyz, inv_two_bw2):
    b2 = jnp.sum(sxyz * sxyz, axis=0)[:, None]
    a2 = jnp.sum(dxyz * dxyz, axis=0)[None, :]
    ab = lax.dot_general(sxyz, dxyz, (((0,), (0,)), ((), ())))
    d2 = jnp.maximum(b2 + a2 - 2.0 * ab, 0.0)

    d2h = d2.astype(jnp.bfloat16)
    cur = jnp.full((1, d2.shape[1]), -jnp.inf, jnp.bfloat16)
    for _ in range(_K):
        cur = jnp.min(jnp.where(d2h > cur, d2h, jnp.inf), axis=0, keepdims=True)

    w = jnp.where(d2h <= cur, jnp.exp(d2 * (-inv_two_bw2)), 0.0)
    return w / (jnp.sum(w, axis=0, keepdims=True) + 1e-8)


def _fused_kernel(l2x_ref, l2p_ref, l3x_ref, l3p_ref, l4x_ref, l4p_ref,
                  W1_ref, b1_ref, W2_ref, b2_ref,
                  W3_ref, b3_ref, W4_ref, b4_ref, out_ref):
    w_a = _interp_weights_t(l4x_ref[0], l3x_ref[0], _INV1)
    w_b = _interp_weights_t(l3x_ref[0], l2x_ref[0], _INV2)

    interp = jnp.dot(l4p_ref[0], w_a)
    new = interp + l3p_ref[0]
    h = jnp.maximum(jnp.dot(W1_ref[...], new) + b1_ref[...], 0.0)
    l3_new = jnp.maximum(jnp.dot(W2_ref[...], h) + b2_ref[...], 0.0)

    interp2 = jnp.dot(l3_new, w_b)
    new2 = interp2 + l2p_ref[0]
    h2 = jnp.maximum(jnp.dot(W3_ref[...], new2) + b3_ref[...], 0.0)
    out_ref[0] = jnp.maximum(jnp.dot(W4_ref[...], h2) + b4_ref[...], 0.0)


def kernel(l1_xyz, l1_points, l2_xyz, l2_points, l3_xyz, l3_points,
           l4_xyz, l4_points, W1, b1, W2, b2, W3, b3, W4, b4):
    del l1_xyz, l1_points
    B = l2_xyz.shape[0]
    Nd2, Nd3, Nd4 = l2_xyz.shape[2], l3_xyz.shape[2], l4_xyz.shape[2]
    C = l2_points.shape[1]
    Cout = W4.shape[0]

    def batch_spec(shape):
        return pl.BlockSpec((1,) + shape, lambda b: (b, 0, 0))

    def full_spec(shape):
        return pl.BlockSpec(shape, lambda b: (0,) * len(shape))

    b1c = b1.reshape(-1, 1)
    b2c = b2.reshape(-1, 1)
    b3c = b3.reshape(-1, 1)
    b4c = b4.reshape(-1, 1)

    out = pl.pallas_call(
        _fused_kernel,
        grid=(B,),
        in_specs=[
            batch_spec((3, Nd2)), batch_spec((C, Nd2)),
            batch_spec((3, Nd3)), batch_spec((C, Nd3)),
            batch_spec((3, Nd4)), batch_spec((C, Nd4)),
            full_spec(W1.shape), full_spec(b1c.shape),
            full_spec(W2.shape), full_spec(b2c.shape),
            full_spec(W3.shape), full_spec(b3c.shape),
            full_spec(W4.shape), full_spec(b4c.shape),
        ],
        out_specs=batch_spec((Cout, Nd2)),
        out_shape=jax.ShapeDtypeStruct((B, Cout, Nd2), l2_points.dtype),
        compiler_params=pltpu.CompilerParams(
            dimension_semantics=("parallel",),
        ),
    )(l2_xyz, l2_points, l3_xyz, l3_points, l4_xyz, l4_points,
      W1, b1c, W2, b2c, W3, b3c, W4, b4c)
    return out
